# Initial kernel scaffold; baseline (speedup 1.0000x reference)
import jax
import jax.numpy as jnp
from jax.experimental import pallas as pl


def kernel(pc, flow):
    raise NotImplementedError("write your pallas kernel here")


# speedup vs baseline: 41.9036x; 41.9036x over previous
import functools

import jax
import jax.numpy as jnp
from jax import lax
from jax.experimental import pallas as pl

_K = 16
_RADIUS = 0.1


def _ballq_body(pc_ref, pct_ref, flow_ref, flowt_ref, out_ref, *, rb, mb, n, k, r2):
    b = pl.program_id(0)
    i = pl.program_id(1)

    @pl.when(jnp.logical_and(b == 0, i == 0))
    def _init():
        out_ref[...] = jnp.zeros((1, 1), jnp.float32)

    prow = pc_ref[0]
    frow = flow_ref[0]
    xr = prow[:, 0:1]
    yr = prow[:, 1:2]
    zr = prow[:, 2:3]
    fxr = frow[:, 0:1]
    fyr = frow[:, 1:2]
    fzr = frow[:, 2:3]

    ii = lax.broadcasted_iota(jnp.int32, (mb, mb), 0)
    jj = lax.broadcasted_iota(jnp.int32, (mb, mb), 1)
    tri = jnp.where(ii < jj, 1.0, 0.0).astype(jnp.float32)

    kf = jnp.float32(k)

    def chunk(j, carry):
        count, acc, padl1 = carry
        sl = pl.ds(j * mb, mb)
        xc = pct_ref[0, 0:1, sl]
        yc = pct_ref[0, 1:2, sl]
        zc = pct_ref[0, 2:3, sl]
        fxc = flowt_ref[0, 0:1, sl]
        fyc = flowt_ref[0, 1:2, sl]
        fzc = flowt_ref[0, 2:3, sl]

        dx = xr - xc
        dy = yr - yc
        dz = zr - zc
        d2 = dx * dx + dy * dy + dz * dz
        w = jnp.where(d2 < r2, 1.0, 0.0).astype(jnp.float32)

        l1 = jnp.abs(fxr - fxc) + jnp.abs(fyr - fyc) + jnp.abs(fzr - fzc)

        prefix = jnp.dot(w, tri, preferred_element_type=jnp.float32) + count
        sel = jnp.where(prefix < kf, w, 0.0)
        first = jnp.where(prefix == 0.0, w, 0.0)

        acc = acc + jnp.sum(sel * l1, axis=1, keepdims=True)
        padl1 = padl1 + jnp.sum(first * l1, axis=1, keepdims=True)
        count = count + jnp.sum(w, axis=1, keepdims=True)
        return count, acc, padl1

    zero = jnp.zeros((rb, 1), jnp.float32)
    count, acc, padl1 = lax.fori_loop(0, n // mb, chunk, (zero, zero, zero))

    pad_n = jnp.maximum(kf - count, 0.0)
    out_ref[...] += jnp.sum(acc + pad_n * padl1, axis=0, keepdims=True)


def kernel(pc, flow):
    b, n, _ = pc.shape
    rb = 512
    mb = 512
    pct = jnp.transpose(pc, (0, 2, 1))
    flowt = jnp.transpose(flow, (0, 2, 1))

    body = functools.partial(
        _ballq_body, rb=rb, mb=mb, n=n, k=_K, r2=_RADIUS * _RADIUS
    )
    total = pl.pallas_call(
        body,
        grid=(b, n // rb),
        in_specs=[
            pl.BlockSpec((1, rb, 3), lambda bb, ii: (bb, ii, 0)),
            pl.BlockSpec((1, 3, n), lambda bb, ii: (bb, 0, 0)),
            pl.BlockSpec((1, rb, 3), lambda bb, ii: (bb, ii, 0)),
            pl.BlockSpec((1, 3, n), lambda bb, ii: (bb, 0, 0)),
        ],
        out_specs=pl.BlockSpec((1, 1), lambda bb, ii: (0, 0)),
        out_shape=jax.ShapeDtypeStruct((1, 1), jnp.float32),
    )(pc, pct, flow, flowt)
    return total[0, 0] / jnp.float32(b * n * _K)


# speedup vs baseline: 43.1082x; 1.0287x over previous
import functools

import jax
import jax.numpy as jnp
from jax import lax
from jax.experimental import pallas as pl

_K = 16
_RADIUS = 0.1


def _ballq_body(pc_ref, pct_ref, flow_ref, flowt_ref, out_ref, *, rb, mb, n, k, r2):
    b = pl.program_id(0)
    i = pl.program_id(1)

    @pl.when(jnp.logical_and(b == 0, i == 0))
    def _init():
        out_ref[...] = jnp.zeros((1, 1), jnp.float32)

    prow = pc_ref[0]
    frow = flow_ref[0]
    xr = jnp.broadcast_to(prow[:, 0:1], (rb, mb))
    yr = jnp.broadcast_to(prow[:, 1:2], (rb, mb))
    zr = jnp.broadcast_to(prow[:, 2:3], (rb, mb))
    fxr = jnp.broadcast_to(frow[:, 0:1], (rb, mb))
    fyr = jnp.broadcast_to(frow[:, 1:2], (rb, mb))
    fzr = jnp.broadcast_to(frow[:, 2:3], (rb, mb))

    ii = lax.broadcasted_iota(jnp.int32, (mb, mb), 0)
    jj = lax.broadcasted_iota(jnp.int32, (mb, mb), 1)
    tri = jnp.where(ii < jj, 1.0, 0.0).astype(jnp.float32)

    kf = jnp.float32(k)

    def chunk(j, carry):
        count, acc, padl1 = carry
        sl = pl.ds(j * mb, mb)
        xc = pct_ref[0, 0:1, sl]
        yc = pct_ref[0, 1:2, sl]
        zc = pct_ref[0, 2:3, sl]
        fxc = flowt_ref[0, 0:1, sl]
        fyc = flowt_ref[0, 1:2, sl]
        fzc = flowt_ref[0, 2:3, sl]

        dx = xr - xc
        dy = yr - yc
        dz = zr - zc
        d2 = dx * dx + dy * dy + dz * dz
        w = jnp.where(d2 < r2, 1.0, 0.0).astype(jnp.float32)

        l1 = jnp.abs(fxr - fxc) + jnp.abs(fyr - fyc) + jnp.abs(fzr - fzc)
        wl1 = w * l1

        prefix = jnp.dot(w, tri, preferred_element_type=jnp.float32) + count
        acc = acc + jnp.where(prefix < kf, wl1, 0.0)
        padl1 = padl1 + jnp.where(prefix == 0.0, wl1, 0.0)
        count = prefix[:, mb - 1:mb] + w[:, mb - 1:mb]
        return count, acc, padl1

    zcol = jnp.zeros((rb, 1), jnp.float32)
    zfull = jnp.zeros((rb, mb), jnp.float32)
    count, acc, padl1 = lax.fori_loop(0, n // mb, chunk, (zcol, zfull, zfull))

    pad_n = jnp.maximum(kf - count, 0.0)
    loss_rows = jnp.sum(acc, axis=1, keepdims=True) + pad_n * jnp.sum(
        padl1, axis=1, keepdims=True
    )
    out_ref[...] += jnp.sum(loss_rows, axis=0, keepdims=True)


def kernel(pc, flow):
    b, n, _ = pc.shape
    rb = 512
    mb = 512
    pct = jnp.transpose(pc, (0, 2, 1))
    flowt = jnp.transpose(flow, (0, 2, 1))

    body = functools.partial(
        _ballq_body, rb=rb, mb=mb, n=n, k=_K, r2=_RADIUS * _RADIUS
    )
    total = pl.pallas_call(
        body,
        grid=(b, n // rb),
        in_specs=[
            pl.BlockSpec((1, rb, 3), lambda bb, ii: (bb, ii, 0)),
            pl.BlockSpec((1, 3, n), lambda bb, ii: (bb, 0, 0)),
            pl.BlockSpec((1, rb, 3), lambda bb, ii: (bb, ii, 0)),
            pl.BlockSpec((1, 3, n), lambda bb, ii: (bb, 0, 0)),
        ],
        out_specs=pl.BlockSpec((1, 1), lambda bb, ii: (0, 0)),
        out_shape=jax.ShapeDtypeStruct((1, 1), jnp.float32),
    )(pc, pct, flow, flowt)
    return total[0, 0] / jnp.float32(b * n * _K)
